# baseline (speedup 1.0000x reference)
import jax
import jax.numpy as jnp
from jax.experimental import pallas as pl

B = 16384
D = 64


def _tc_loss_body(x_ref, g_ref, out_ref):
    i = pl.program_id(0)
    x = x_ref[...]
    g = g_ref[...]
    m = jnp.max(x, axis=1, keepdims=True)
    e = jnp.exp(x - m)
    z = jnp.sum(e, axis=1, keepdims=True)
    logsm = x - m - jnp.log(z)
    part = -jnp.sum(logsm * g, keepdims=True) * (1.0 / B)

    @pl.when(i == 0)
    def _init():
        out_ref[...] = part

    @pl.when(i != 0)
    def _acc():
        out_ref[...] += part


_N_BLK = 8
_BLK = B // _N_BLK

_tc_loss = pl.pallas_call(
    _tc_loss_body,
    grid=(_N_BLK,),
    in_specs=[
        pl.BlockSpec((_BLK, D), lambda i: (i, 0)),
        pl.BlockSpec((_BLK, D), lambda i: (i, 0)),
    ],
    out_specs=pl.BlockSpec((1, 1), lambda i: (0, 0)),
    out_shape=jax.ShapeDtypeStruct((1, 1), jnp.float32),
)


def kernel(outputs, index, confidence):
    gathered = jnp.take(confidence, index, axis=0)
    loss = _tc_loss(outputs, gathered)
    return loss[0, 0]
